# baseline (speedup 1.0000x reference)
import functools

import jax
import jax.numpy as jnp
from jax import lax
from jax.experimental import pallas as pl
from jax.experimental.pallas import tpu as pltpu
from jax.experimental.pallas import tpu_sc as plsc

D = 128
NC, NS = 2, 16
NW = NC * NS
CHUNK = 64
NBUF = 3
R = 1280


def _dense_body(x_ref, w_ref, b_ref, d0_ref, d1_ref, hs_ref, dinv_ref, si_ref):
    h = jnp.dot(x_ref[...], w_ref[...], preferred_element_type=jnp.float32)
    deg = d0_ref[...] + d1_ref[...] + 1.0
    dinv = lax.rsqrt(jnp.maximum(deg, 1.0))
    hs_ref[...] = h * dinv
    dinv_ref[...] = dinv
    si_ref[...] = h * (dinv * dinv) + b_ref[...]


def _final_body(a0_ref, a1_ref, dinv_ref, si_ref, out_ref):
    out_ref[...] = dinv_ref[...] * (a0_ref[...] + a1_ref[...]) + si_ref[...]


def _unpack_chunk(packed_v, j, sidx_v, didx_v, p, col0=0):
    def ub(k, _):
        v = packed_v[j, pl.ds(k * 16, 16)]
        if sidx_v is not None:
            sidx_v[p, pl.ds(col0 + k * 16, 16)] = v & 0xFFFF
        didx_v[p, pl.ds(col0 + k * 16, 16)] = lax.shift_right_logical(v, 16)
        return 0

    lax.fori_loop(0, CHUNK // 16, ub, 0)


def _make_deg_body(ct):
    def _deg_body(packed_hbm, ones_hbm, deg_out,
                  packed_v, didx_v, ones_v, z_v, deg_sh):
        c = lax.axis_index("c")
        s = lax.axis_index("s")
        wid = s * NC + c
        n_pad = deg_sh.shape[0]
        sl = n_pad // NS
        z16 = jnp.zeros((16,), jnp.float32)

        def zero_body(i, _):
            z_v[pl.ds(i * 16, 16)] = z16
            return 0

        lax.fori_loop(0, sl // 16, zero_body, 0)
        pltpu.sync_copy(z_v, deg_sh.at[pl.ds(s * sl, sl)])
        plsc.subcore_barrier()

        pltpu.sync_copy(ones_hbm, ones_v)
        pltpu.sync_copy(packed_hbm.at[wid], packed_v)

        def add_body(q, _):
            _unpack_chunk(packed_v, 2 * q, None, didx_v, 0, col0=0)
            _unpack_chunk(packed_v, 2 * q + 1, None, didx_v, 0, col0=CHUNK)
            pltpu.sync_copy(ones_v, deg_sh.at[didx_v.at[0]], add=True)
            return 0

        lax.fori_loop(0, ct // 2, add_body, 0)
        plsc.subcore_barrier()
        pltpu.sync_copy(deg_sh.at[pl.ds(s * sl, sl)], deg_out.at[c, s])

    return _deg_body


def _make_msg_body(ct):
    def _msg_body(hs_hbm, packed_hbm, acc_out,
                  packed_v, sidx_v, didx_v, rows_list, acc_sh,
                  gsems, ssems):
        c = lax.axis_index("c")
        s = lax.axis_index("s")
        wid = s * NC + c
        n_pad = acc_sh.shape[0]
        rl = n_pad // NS
        z16 = jnp.zeros((16,), jnp.float32)
        rows0_v = rows_list[0]

        def zrow_body(i, _):
            for k in range(D // 16):
                rows0_v[i, pl.ds(k * 16, 16)] = z16
            return 0

        lax.fori_loop(0, CHUNK, zrow_body, 0)

        def zcopy_body(q, _):
            pltpu.sync_copy(rows0_v, acc_sh.at[pl.ds(s * rl + q * CHUNK, CHUNK)])
            return 0

        lax.fori_loop(0, rl // CHUNK, zcopy_body, 0)
        plsc.subcore_barrier()

        pltpu.sync_copy(packed_hbm.at[wid], packed_v)

        def gather(j, p):
            _unpack_chunk(packed_v, j, sidx_v, didx_v, p)
            pltpu.async_copy(hs_hbm.at[sidx_v.at[p]], rows_list[p], gsems[p])

        def gwait(p):
            pltpu.make_async_copy(hs_hbm.at[sidx_v.at[p]], rows_list[p],
                                  gsems[p]).wait()

        def sfire(p):
            pltpu.async_copy(rows_list[p], acc_sh.at[didx_v.at[p]], ssems[p],
                             add=True)

        def swait(p):
            pltpu.make_async_copy(rows_list[p], acc_sh.at[didx_v.at[p]],
                                  ssems[p]).wait()

        gather(0, 0)
        gather(1, 1)

        def ring_body(q, _):
            for k in range(NBUF):
                j = NBUF * q + k
                nx = (k + 2) % NBUF
                gwait(k)
                sfire(k)
                if k >= NBUF - 2:
                    swait(nx)
                else:
                    @pl.when(j >= NBUF - 2)
                    def _():
                        swait(nx)

                @pl.when(j + 2 < ct)
                def _():
                    gather(j + 2, nx)

            return 0

        lax.fori_loop(0, ct // NBUF, ring_body, 0)
        for t in range(NBUF - 2):
            swait((2 + t) % NBUF)
        plsc.subcore_barrier()
        pltpu.sync_copy(acc_sh.at[pl.ds(s * rl, rl)], acc_out.at[c, s])

    return _msg_body


def kernel(x, edge_index, W, b):
    n = x.shape[0]
    e = edge_index.shape[1]
    n_pad = -(-(n + 1) // R) * R
    sl = n_pad // NS
    ct = -(-e // (CHUNK * NW))
    ct = -(-ct // NBUF) * NBUF
    trash = n
    grid = n_pad // R

    src = edge_index[0].astype(jnp.int32)
    dst = edge_index[1].astype(jnp.int32)
    e_slots = ct * NW * CHUNK
    pad = e_slots - e
    src_p = jnp.concatenate([src, jnp.zeros((pad,), jnp.int32)])
    dst_p = jnp.concatenate([dst, jnp.full((pad,), trash, jnp.int32)])
    packed = (src_p | (dst_p << 16)).reshape(NW, ct, CHUNK)

    x_pad = jnp.pad(x, ((0, n_pad - n), (0, 0)))
    ones = jnp.ones((2 * CHUNK,), jnp.float32)
    b2 = b.reshape(1, D).astype(jnp.float32)

    mesh = plsc.VectorSubcoreMesh(core_axis_name="c", subcore_axis_name="s",
                                  num_cores=NC, num_subcores=NS)

    deg_kernel = functools.partial(
        pl.kernel,
        out_type=jax.ShapeDtypeStruct((NC, NS, sl), jnp.float32),
        mesh=mesh,
        scratch_types=[
            pltpu.VMEM((ct, CHUNK), jnp.int32),
            pltpu.VMEM((1, 2 * CHUNK), jnp.int32),
            pltpu.VMEM((2 * CHUNK,), jnp.float32),
            pltpu.VMEM((sl,), jnp.float32),
            pltpu.VMEM_SHARED((n_pad,), jnp.float32),
        ],
    )(_make_deg_body(ct))
    deg_p = deg_kernel(packed, ones).reshape(NC, n_pad)

    deg0 = deg_p[0].reshape(n_pad, 1)
    deg1 = deg_p[1].reshape(n_pad, 1)

    hs, dinv, selfinit = pl.pallas_call(
        _dense_body,
        grid=(grid,),
        in_specs=[
            pl.BlockSpec((R, D), lambda i: (i, 0)),
            pl.BlockSpec((D, D), lambda i: (0, 0)),
            pl.BlockSpec((1, D), lambda i: (0, 0)),
            pl.BlockSpec((R, 1), lambda i: (i, 0)),
            pl.BlockSpec((R, 1), lambda i: (i, 0)),
        ],
        out_specs=[
            pl.BlockSpec((R, D), lambda i: (i, 0)),
            pl.BlockSpec((R, 1), lambda i: (i, 0)),
            pl.BlockSpec((R, D), lambda i: (i, 0)),
        ],
        out_shape=[
            jax.ShapeDtypeStruct((n_pad, D), jnp.float32),
            jax.ShapeDtypeStruct((n_pad, 1), jnp.float32),
            jax.ShapeDtypeStruct((n_pad, D), jnp.float32),
        ],
    )(x_pad, W.astype(jnp.float32), b2, deg0, deg1)

    msg_kernel = functools.partial(
        pl.kernel,
        out_type=jax.ShapeDtypeStruct((NC, NS, sl, D), jnp.float32),
        mesh=mesh,
        scratch_types=[
            pltpu.VMEM((ct, CHUNK), jnp.int32),
            pltpu.VMEM((NBUF, CHUNK), jnp.int32),
            pltpu.VMEM((NBUF, CHUNK), jnp.int32),
            [pltpu.VMEM((CHUNK, D), jnp.float32) for _ in range(NBUF)],
            pltpu.VMEM_SHARED((n_pad, D), jnp.float32),
            [pltpu.SemaphoreType.DMA for _ in range(NBUF)],
            [pltpu.SemaphoreType.DMA for _ in range(NBUF)],
        ],
    )(_make_msg_body(ct))
    acc_p = msg_kernel(hs, packed).reshape(NC, n_pad, D)

    out = pl.pallas_call(
        _final_body,
        grid=(grid,),
        in_specs=[
            pl.BlockSpec((R, D), lambda i: (i, 0)),
            pl.BlockSpec((R, D), lambda i: (i, 0)),
            pl.BlockSpec((R, 1), lambda i: (i, 0)),
            pl.BlockSpec((R, D), lambda i: (i, 0)),
        ],
        out_specs=pl.BlockSpec((R, D), lambda i: (i, 0)),
        out_shape=jax.ShapeDtypeStruct((n_pad, D), jnp.float32),
    )(acc_p[0], acc_p[1], dinv, selfinit)

    return out[:n]


# speedup vs baseline: 1.0926x; 1.0926x over previous
import functools

import jax
import jax.numpy as jnp
from jax import lax
from jax.experimental import pallas as pl
from jax.experimental.pallas import tpu as pltpu
from jax.experimental.pallas import tpu_sc as plsc

D = 128
NC, NS = 2, 16
NW = NC * NS
CHUNK = 128
R = 1280


def _dense_body(x_ref, w_ref, b_ref, d0_ref, d1_ref, hs_ref, dinv_ref, si_ref):
    h = jnp.dot(x_ref[...], w_ref[...], preferred_element_type=jnp.float32)
    deg = d0_ref[...] + d1_ref[...] + 1.0
    dinv = lax.rsqrt(jnp.maximum(deg, 1.0))
    hs_ref[...] = h * dinv
    dinv_ref[...] = dinv
    si_ref[...] = h * (dinv * dinv) + b_ref[...]


def _final_body(a0_ref, a1_ref, dinv_ref, si_ref, out_ref):
    out_ref[...] = dinv_ref[...] * (a0_ref[...] + a1_ref[...]) + si_ref[...]


def _unpack_chunk(packed_v, j, sidx_v, didx_v, p):
    def ub(k, _):
        v = packed_v[j, pl.ds(k * 16, 16)]
        if sidx_v is not None:
            sidx_v[p, pl.ds(k * 16, 16)] = v & 0xFFFF
        didx_v[p, pl.ds(k * 16, 16)] = lax.shift_right_logical(v, 16)
        return 0

    lax.fori_loop(0, CHUNK // 16, ub, 0)


def _make_deg_body(ct):
    def _deg_body(packed_hbm, ones_hbm, deg_out,
                  packed_v, didx_v, ones_v, z_v, deg_sh):
        c = lax.axis_index("c")
        s = lax.axis_index("s")
        wid = s * NC + c
        n_pad = deg_sh.shape[0]
        sl = n_pad // NS
        z16 = jnp.zeros((16,), jnp.float32)

        def zero_body(i, _):
            z_v[pl.ds(i * 16, 16)] = z16
            return 0

        lax.fori_loop(0, sl // 16, zero_body, 0)
        pltpu.sync_copy(z_v, deg_sh.at[pl.ds(s * sl, sl)])
        plsc.subcore_barrier()

        pltpu.sync_copy(ones_hbm, ones_v)
        pltpu.sync_copy(packed_hbm.at[wid], packed_v)

        def add_body(j, _):
            _unpack_chunk(packed_v, j, None, didx_v, 0)
            pltpu.sync_copy(ones_v, deg_sh.at[didx_v.at[0]], add=True)
            return 0

        lax.fori_loop(0, ct, add_body, 0)
        plsc.subcore_barrier()
        pltpu.sync_copy(deg_sh.at[pl.ds(s * sl, sl)], deg_out.at[c, s])

    return _deg_body


def _make_msg_body(ct):
    def _msg_body(hs_hbm, packed_hbm, acc_out,
                  packed_v, sidx_v, didx_v, rows0_v, rows1_v, acc_sh,
                  sem0, sem1):
        c = lax.axis_index("c")
        s = lax.axis_index("s")
        wid = s * NC + c
        n_pad = acc_sh.shape[0]
        rl = n_pad // NS
        z16 = jnp.zeros((16,), jnp.float32)

        def zrow_body(i, _):
            for k in range(D // 16):
                rows0_v[i, pl.ds(k * 16, 16)] = z16
            return 0

        lax.fori_loop(0, CHUNK, zrow_body, 0)

        def zcopy_body(q, _):
            pltpu.sync_copy(rows0_v, acc_sh.at[pl.ds(s * rl + q * CHUNK, CHUNK)])
            return 0

        lax.fori_loop(0, rl // CHUNK, zcopy_body, 0)
        plsc.subcore_barrier()

        pltpu.sync_copy(packed_hbm.at[wid], packed_v)

        def gather(p, buf, sem):
            pltpu.async_copy(hs_hbm.at[sidx_v.at[p]], buf, sem)

        def gwait(p, buf, sem):
            pltpu.make_async_copy(hs_hbm.at[sidx_v.at[p]], buf, sem).wait()

        def scatter(p, buf):
            pltpu.sync_copy(buf, acc_sh.at[didx_v.at[p]], add=True)

        _unpack_chunk(packed_v, 0, sidx_v, didx_v, 0)
        gather(0, rows0_v, sem0)

        def edge_body(q, _):
            j1 = 2 * q + 1
            _unpack_chunk(packed_v, j1, sidx_v, didx_v, 1)
            gwait(0, rows0_v, sem0)
            gather(1, rows1_v, sem1)
            scatter(0, rows0_v)

            @pl.when(j1 + 1 < ct)
            def _():
                _unpack_chunk(packed_v, j1 + 1, sidx_v, didx_v, 0)

            gwait(1, rows1_v, sem1)

            @pl.when(j1 + 1 < ct)
            def _():
                gather(0, rows0_v, sem0)

            scatter(1, rows1_v)
            return 0

        lax.fori_loop(0, ct // 2, edge_body, 0)

        if ct % 2 == 1:
            gwait(0, rows0_v, sem0)
            scatter(0, rows0_v)

        plsc.subcore_barrier()
        pltpu.sync_copy(acc_sh.at[pl.ds(s * rl, rl)], acc_out.at[c, s])

    return _msg_body


def kernel(x, edge_index, W, b):
    n = x.shape[0]
    e = edge_index.shape[1]
    n_pad = -(-(n + 1) // R) * R
    sl = n_pad // NS
    ct = -(-e // (CHUNK * NW))
    trash = n
    grid = n_pad // R

    src = edge_index[0].astype(jnp.int32)
    dst = edge_index[1].astype(jnp.int32)
    e_slots = ct * NW * CHUNK
    pad = e_slots - e
    src_p = jnp.concatenate([src, jnp.zeros((pad,), jnp.int32)])
    dst_p = jnp.concatenate([dst, jnp.full((pad,), trash, jnp.int32)])
    packed = (src_p | (dst_p << 16)).reshape(NW, ct, CHUNK)

    x_pad = jnp.pad(x, ((0, n_pad - n), (0, 0)))
    ones = jnp.ones((CHUNK,), jnp.float32)
    b2 = b.reshape(1, D).astype(jnp.float32)

    mesh = plsc.VectorSubcoreMesh(core_axis_name="c", subcore_axis_name="s",
                                  num_cores=NC, num_subcores=NS)

    deg_kernel = functools.partial(
        pl.kernel,
        out_type=jax.ShapeDtypeStruct((NC, NS, sl), jnp.float32),
        mesh=mesh,
        scratch_types=[
            pltpu.VMEM((ct, CHUNK), jnp.int32),
            pltpu.VMEM((1, CHUNK), jnp.int32),
            pltpu.VMEM((CHUNK,), jnp.float32),
            pltpu.VMEM((sl,), jnp.float32),
            pltpu.VMEM_SHARED((n_pad,), jnp.float32),
        ],
    )(_make_deg_body(ct))
    deg_p = deg_kernel(packed, ones).reshape(NC, n_pad)

    deg0 = deg_p[0].reshape(n_pad, 1)
    deg1 = deg_p[1].reshape(n_pad, 1)

    hs, dinv, selfinit = pl.pallas_call(
        _dense_body,
        grid=(grid,),
        in_specs=[
            pl.BlockSpec((R, D), lambda i: (i, 0)),
            pl.BlockSpec((D, D), lambda i: (0, 0)),
            pl.BlockSpec((1, D), lambda i: (0, 0)),
            pl.BlockSpec((R, 1), lambda i: (i, 0)),
            pl.BlockSpec((R, 1), lambda i: (i, 0)),
        ],
        out_specs=[
            pl.BlockSpec((R, D), lambda i: (i, 0)),
            pl.BlockSpec((R, 1), lambda i: (i, 0)),
            pl.BlockSpec((R, D), lambda i: (i, 0)),
        ],
        out_shape=[
            jax.ShapeDtypeStruct((n_pad, D), jnp.float32),
            jax.ShapeDtypeStruct((n_pad, 1), jnp.float32),
            jax.ShapeDtypeStruct((n_pad, D), jnp.float32),
        ],
    )(x_pad, W.astype(jnp.float32), b2, deg0, deg1)

    msg_kernel = functools.partial(
        pl.kernel,
        out_type=jax.ShapeDtypeStruct((NC, NS, sl, D), jnp.float32),
        mesh=mesh,
        scratch_types=[
            pltpu.VMEM((ct, CHUNK), jnp.int32),
            pltpu.VMEM((2, CHUNK), jnp.int32),
            pltpu.VMEM((2, CHUNK), jnp.int32),
            pltpu.VMEM((CHUNK, D), jnp.float32),
            pltpu.VMEM((CHUNK, D), jnp.float32),
            pltpu.VMEM_SHARED((n_pad, D), jnp.float32),
            pltpu.SemaphoreType.DMA,
            pltpu.SemaphoreType.DMA,
        ],
    )(_make_msg_body(ct))
    acc_p = msg_kernel(hs, packed).reshape(NC, n_pad, D)

    out = pl.pallas_call(
        _final_body,
        grid=(grid,),
        in_specs=[
            pl.BlockSpec((R, D), lambda i: (i, 0)),
            pl.BlockSpec((R, D), lambda i: (i, 0)),
            pl.BlockSpec((R, 1), lambda i: (i, 0)),
            pl.BlockSpec((R, D), lambda i: (i, 0)),
        ],
        out_specs=pl.BlockSpec((R, D), lambda i: (i, 0)),
        out_shape=jax.ShapeDtypeStruct((n_pad, D), jnp.float32),
    )(acc_p[0], acc_p[1], dinv, selfinit)

    return out[:n]


# speedup vs baseline: 1.4444x; 1.3219x over previous
import functools

import jax
import jax.numpy as jnp
from jax import lax
from jax.experimental import pallas as pl
from jax.experimental.pallas import tpu as pltpu
from jax.experimental.pallas import tpu_sc as plsc

D = 128
NC, NS = 2, 16
NW = NC * NS
CHUNK = 128
R = 1280
FRAC0 = 0.5


def _dense_body(x_ref, w_ref, b_ref, d0_ref, d1_ref, hs_ref, dinv_ref, si_ref):
    h = jnp.dot(x_ref[...], w_ref[...], preferred_element_type=jnp.float32)
    deg = d0_ref[...] + d1_ref[...] + 1.0
    dinv = lax.rsqrt(jnp.maximum(deg, 1.0))
    hs_ref[...] = h * dinv
    dinv_ref[...] = dinv
    si_ref[...] = h * (dinv * dinv) + b_ref[...]


def _final_body(a0_ref, a1_ref, dinv_ref, si_ref, out_ref):
    out_ref[...] = dinv_ref[...] * (a0_ref[...] + a1_ref[...]) + si_ref[...]


def _unpack_chunk(packed_v, j, sidx_v, didx_v, p):
    def ub(k, _):
        v = packed_v[j, pl.ds(k * 16, 16)]
        if sidx_v is not None:
            sidx_v[p, pl.ds(k * 16, 16)] = v & 0xFFFF
        didx_v[p, pl.ds(k * 16, 16)] = lax.shift_right_logical(v, 16)
        return 0

    lax.fori_loop(0, CHUNK // 16, ub, 0)


def _make_deg_body(ct0, ct1):
    def _deg_body(packed_hbm, ones_hbm, deg_out,
                  packed_v, didx_v, ones_v, z_v, deg_sh):
        c = lax.axis_index("c")
        s = lax.axis_index("s")
        wid = s * NC + c
        myct = jnp.where(c == 0, ct0, ct1)
        n_pad = deg_sh.shape[0]
        sl = n_pad // NS
        z16 = jnp.zeros((16,), jnp.float32)

        def zero_body(i, _):
            z_v[pl.ds(i * 16, 16)] = z16
            return 0

        lax.fori_loop(0, sl // 16, zero_body, 0)
        pltpu.sync_copy(z_v, deg_sh.at[pl.ds(s * sl, sl)])
        plsc.subcore_barrier()

        pltpu.sync_copy(ones_hbm, ones_v)
        pltpu.sync_copy(packed_hbm.at[wid], packed_v)

        def add_body(j, _):
            _unpack_chunk(packed_v, j, None, didx_v, 0)
            pltpu.sync_copy(ones_v, deg_sh.at[didx_v.at[0]], add=True)
            return 0

        lax.fori_loop(0, myct, add_body, 0)
        plsc.subcore_barrier()
        pltpu.sync_copy(deg_sh.at[pl.ds(s * sl, sl)], deg_out.at[c, s])

    return _deg_body


def _make_msg_body(ct0, ct1):
    def _msg_body(hs_hbm, packed_hbm, acc_out,
                  packed_v, sidx_v, didx_v, rows0_v, rows1_v, acc_sh,
                  sem0, sem1):
        c = lax.axis_index("c")
        s = lax.axis_index("s")
        wid = s * NC + c
        myct = jnp.where(c == 0, ct0, ct1)
        n_pad = acc_sh.shape[0]
        rl = n_pad // NS
        z16 = jnp.zeros((16,), jnp.float32)

        def zrow_body(i, _):
            for k in range(D // 16):
                rows0_v[i, pl.ds(k * 16, 16)] = z16
            return 0

        lax.fori_loop(0, CHUNK, zrow_body, 0)

        def zcopy_body(q, _):
            pltpu.sync_copy(rows0_v, acc_sh.at[pl.ds(s * rl + q * CHUNK, CHUNK)])
            return 0

        lax.fori_loop(0, rl // CHUNK, zcopy_body, 0)
        plsc.subcore_barrier()

        pltpu.sync_copy(packed_hbm.at[wid], packed_v)

        def gather(p, buf, sem):
            pltpu.async_copy(hs_hbm.at[sidx_v.at[p]], buf, sem)

        def gwait(p, buf, sem):
            pltpu.make_async_copy(hs_hbm.at[sidx_v.at[p]], buf, sem).wait()

        def scatter(p, buf):
            pltpu.sync_copy(buf, acc_sh.at[didx_v.at[p]], add=True)

        _unpack_chunk(packed_v, 0, sidx_v, didx_v, 0)
        gather(0, rows0_v, sem0)

        def edge_body(q, _):
            j1 = 2 * q + 1
            _unpack_chunk(packed_v, j1, sidx_v, didx_v, 1)
            gwait(0, rows0_v, sem0)
            gather(1, rows1_v, sem1)
            scatter(0, rows0_v)

            @pl.when(j1 + 1 < myct)
            def _():
                _unpack_chunk(packed_v, j1 + 1, sidx_v, didx_v, 0)

            gwait(1, rows1_v, sem1)

            @pl.when(j1 + 1 < myct)
            def _():
                gather(0, rows0_v, sem0)

            scatter(1, rows1_v)
            return 0

        lax.fori_loop(0, myct // 2, edge_body, 0)

        @pl.when(myct % 2 == 1)
        def _():
            gwait(0, rows0_v, sem0)
            scatter(0, rows0_v)

        plsc.subcore_barrier()
        pltpu.sync_copy(acc_sh.at[pl.ds(s * rl, rl)], acc_out.at[c, s])

    return _msg_body


def kernel(x, edge_index, W, b):
    n = x.shape[0]
    e = edge_index.shape[1]
    n_pad = -(-(n + 1) // R) * R
    sl = n_pad // NS
    cpt = -(-e // (CHUNK * NS))
    ct0 = min(max(1, round(cpt * FRAC0)), cpt - 1)
    ct1 = cpt - ct0
    ct_max = max(ct0, ct1)
    trash = n
    grid = n_pad // R

    src = edge_index[0].astype(jnp.int32)
    dst = edge_index[1].astype(jnp.int32)
    e_slots = cpt * NS * CHUNK
    pad = e_slots - e
    trash_packed = trash << 16
    src_p = jnp.concatenate([src, jnp.zeros((pad,), jnp.int32)])
    dst_p = jnp.concatenate([dst, jnp.full((pad,), trash, jnp.int32)])
    packed_flat = src_p | (dst_p << 16)
    blocks = []
    off = 0
    for w in range(NW):
        cw = ct0 if w % 2 == 0 else ct1
        blk = packed_flat[off * CHUNK:(off + cw) * CHUNK].reshape(cw, CHUNK)
        off += cw
        if cw < ct_max:
            blk = jnp.pad(blk, ((0, ct_max - cw), (0, 0)),
                          constant_values=trash_packed)
        blocks.append(blk)
    packed = jnp.stack(blocks)

    x_pad = jnp.pad(x, ((0, n_pad - n), (0, 0)))
    ones = jnp.ones((CHUNK,), jnp.float32)
    b2 = b.reshape(1, D).astype(jnp.float32)

    mesh = plsc.VectorSubcoreMesh(core_axis_name="c", subcore_axis_name="s",
                                  num_cores=NC, num_subcores=NS)

    deg_kernel = functools.partial(
        pl.kernel,
        out_type=jax.ShapeDtypeStruct((NC, NS, sl), jnp.float32),
        mesh=mesh,
        scratch_types=[
            pltpu.VMEM((ct_max, CHUNK), jnp.int32),
            pltpu.VMEM((1, CHUNK), jnp.int32),
            pltpu.VMEM((CHUNK,), jnp.float32),
            pltpu.VMEM((sl,), jnp.float32),
            pltpu.VMEM_SHARED((n_pad,), jnp.float32),
        ],
    )(_make_deg_body(ct0, ct1))
    deg_p = deg_kernel(packed, ones).reshape(NC, n_pad)

    deg0 = deg_p[0].reshape(n_pad, 1)
    deg1 = deg_p[1].reshape(n_pad, 1)

    hs, dinv, selfinit = pl.pallas_call(
        _dense_body,
        grid=(grid,),
        in_specs=[
            pl.BlockSpec((R, D), lambda i: (i, 0)),
            pl.BlockSpec((D, D), lambda i: (0, 0)),
            pl.BlockSpec((1, D), lambda i: (0, 0)),
            pl.BlockSpec((R, 1), lambda i: (i, 0)),
            pl.BlockSpec((R, 1), lambda i: (i, 0)),
        ],
        out_specs=[
            pl.BlockSpec((R, D), lambda i: (i, 0)),
            pl.BlockSpec((R, 1), lambda i: (i, 0)),
            pl.BlockSpec((R, D), lambda i: (i, 0)),
        ],
        out_shape=[
            jax.ShapeDtypeStruct((n_pad, D), jnp.float32),
            jax.ShapeDtypeStruct((n_pad, 1), jnp.float32),
            jax.ShapeDtypeStruct((n_pad, D), jnp.float32),
        ],
    )(x_pad, W.astype(jnp.float32), b2, deg0, deg1)

    msg_kernel = functools.partial(
        pl.kernel,
        out_type=jax.ShapeDtypeStruct((NC, NS, sl, D), jnp.float32),
        mesh=mesh,
        scratch_types=[
            pltpu.VMEM((ct_max, CHUNK), jnp.int32),
            pltpu.VMEM((2, CHUNK), jnp.int32),
            pltpu.VMEM((2, CHUNK), jnp.int32),
            pltpu.VMEM((CHUNK, D), jnp.float32),
            pltpu.VMEM((CHUNK, D), jnp.float32),
            pltpu.VMEM_SHARED((n_pad, D), jnp.float32),
            pltpu.SemaphoreType.DMA,
            pltpu.SemaphoreType.DMA,
        ],
    )(_make_msg_body(ct0, ct1))
    acc_p = msg_kernel(hs, packed).reshape(NC, n_pad, D)

    out = pl.pallas_call(
        _final_body,
        grid=(grid,),
        in_specs=[
            pl.BlockSpec((R, D), lambda i: (i, 0)),
            pl.BlockSpec((R, D), lambda i: (i, 0)),
            pl.BlockSpec((R, 1), lambda i: (i, 0)),
            pl.BlockSpec((R, D), lambda i: (i, 0)),
        ],
        out_specs=pl.BlockSpec((R, D), lambda i: (i, 0)),
        out_shape=jax.ShapeDtypeStruct((n_pad, D), jnp.float32),
    )(acc_p[0], acc_p[1], dinv, selfinit)

    return out[:n]
